# Initial kernel scaffold; baseline (speedup 1.0000x reference)
import jax
import jax.numpy as jnp
from jax import lax
from jax.experimental import pallas as pl
from jax.experimental.pallas import tpu as pltpu
from jax.experimental.pallas import tpu_sc as plsc

_N = 10000
_E = 160000
_D = 256
_DH = 128
_L = 6
_N_PAD = 10240
_E_PAD = 163840
_EPT = _E_PAD // 16
_CH = 512
_NCH = _EPT // _CH
_RPT = _N_PAD // 16


def _seg_mesh():
  return plsc.VectorSubcoreMesh(core_axis_name="c", subcore_axis_name="s")


def _sc_deg_body(dst_hbm, out_hbm, hist, ebuf, rbuf, obuf, stage):
  c = lax.axis_index("c")
  sid = lax.axis_index("s")
  zeros16 = jnp.zeros((16,), jnp.float32)
  ones16 = jnp.ones((16,), jnp.float32)

  @pl.when(c == 0)
  def _():
    def zloop(i, _):
      hist[pl.ds(i * 16, 16)] = zeros16
      return ()
    lax.fori_loop(0, _N_PAD // 16, zloop, ())

    n_load = _EPT // 2048
    def chunk(k, _):
      pltpu.sync_copy(dst_hbm.at[pl.ds(sid * (_EPT // 128) + k * 16, 16)],
                      ebuf)
      def grp(g, _):
        idx = ebuf[g // 8, pl.ds((g % 8) * 16, 16)]
        plsc.addupdate_scatter(hist, [idx], ones16)
        return ()
      lax.fori_loop(0, 128, grp, ())
      return ()
    lax.fori_loop(0, n_load, chunk, ())

    pltpu.sync_copy(hist, stage.at[sid])
    plsc.subcore_barrier()

    base = sid * _RPT
    for r in range(16):
      pltpu.sync_copy(stage.at[r, pl.ds(base, _RPT)], rbuf.at[r])

    def rloop(g, _):
      v = rbuf[0, pl.ds(g * 16, 16)]
      for r in range(1, 16):
        v = v + rbuf[r, pl.ds(g * 16, 16)]
      obuf[pl.ds(g * 16, 16)] = 1.0 / jnp.maximum(v, 1.0)
      return ()
    lax.fori_loop(0, _RPT // 16, rloop, ())
    pltpu.sync_copy(obuf, out_hbm.at[pl.ds(base, _RPT)])


@jax.jit
def _sc_deg(dst2d):
  return pl.kernel(
      _sc_deg_body,
      out_type=jax.ShapeDtypeStruct((_N_PAD,), jnp.float32),
      mesh=_seg_mesh(),
      scratch_types=[
          pltpu.VMEM((_N_PAD,), jnp.float32),
          pltpu.VMEM((16, 128), jnp.int32),
          pltpu.VMEM((16, _RPT), jnp.float32),
          pltpu.VMEM((_RPT,), jnp.float32),
          pltpu.VMEM_SHARED((16, _N_PAD), jnp.float32),
      ],
  )(dst2d)


def _sc_segsum_body(t0_hbm, t1_hbm, src_hbm, dst_hbm, s0_hbm, s1_hbm,
                    acc, rows, sidx, didx, sem):
  c = lax.axis_index("c")
  sid = lax.axis_index("s")
  zeros16 = jnp.zeros((16,), jnp.float32)

  def zloop(i, _):
    for j in range(8):
      rows[i, pl.ds(j * 16, 16)] = zeros16
    return ()
  lax.fori_loop(0, _CH, zloop, ())
  pltpu.sync_copy(rows, acc.at[pl.ds(sid * _RPT, _CH)])
  pltpu.sync_copy(rows.at[pl.ds(0, _RPT - _CH)],
                  acc.at[pl.ds(sid * _RPT + _CH, _RPT - _CH)])
  plsc.subcore_barrier()

  def edge_pass(t_hbm):
    def chunk(k, _):
      rowbase = (sid * _EPT + k * _CH) // 128
      pltpu.sync_copy(src_hbm.at[pl.ds(rowbase, _CH // 128)], sidx)
      pltpu.sync_copy(dst_hbm.at[pl.ds(rowbase, _CH // 128)], didx)
      pltpu.async_copy(t_hbm.at[sidx], rows.at[pl.ds(0, _CH)], sem).wait()
      pltpu.sync_copy(rows.at[pl.ds(0, _CH)], acc.at[didx], add=True)
      return ()
    lax.fori_loop(0, _NCH, chunk, ())

  @pl.when(c == 0)
  def _():
    edge_pass(t0_hbm)

  @pl.when(c == 1)
  def _():
    edge_pass(t1_hbm)

  plsc.subcore_barrier()

  @pl.when(c == 0)
  def _():
    pltpu.sync_copy(acc.at[pl.ds(sid * _RPT, _RPT)],
                    s0_hbm.at[pl.ds(sid * _RPT, _RPT)])

  @pl.when(c == 1)
  def _():
    pltpu.sync_copy(acc.at[pl.ds(sid * _RPT, _RPT)],
                    s1_hbm.at[pl.ds(sid * _RPT, _RPT)])


@jax.jit
def _sc_segsum(t0, t1, src2d, dst2d):
  return pl.kernel(
      _sc_segsum_body,
      out_type=(jax.ShapeDtypeStruct((_N_PAD, _DH), jnp.float32),
                jax.ShapeDtypeStruct((_N_PAD, _DH), jnp.float32)),
      mesh=_seg_mesh(),
      scratch_types=[
          pltpu.VMEM_SHARED((_N_PAD, _DH), jnp.float32),
          pltpu.VMEM((_CH, _DH), jnp.float32),
          pltpu.VMEM((_CH // 128, 128), jnp.int32),
          pltpu.VMEM((_CH // 128, 128), jnp.int32),
          pltpu.SemaphoreType.DMA,
      ],
  )(t0, t1, src2d, dst2d)


def _tc_mm0_body(x_ref, w_ref, o0_ref, o1_ref):
  t = jnp.dot(x_ref[...], w_ref[...], preferred_element_type=jnp.float32)
  o0_ref[...] = t[:, :_DH]
  o1_ref[...] = t[:, _DH:]


@jax.jit
def _tc_mm0(x, w):
  blk = 2000
  return pl.pallas_call(
      _tc_mm0_body,
      grid=(_N // blk,),
      in_specs=[
          pl.BlockSpec((blk, _D), lambda i: (i, 0)),
          pl.BlockSpec((_D, _D), lambda i: (0, 0)),
      ],
      out_specs=(
          pl.BlockSpec((blk, _DH), lambda i: (i, 0)),
          pl.BlockSpec((blk, _DH), lambda i: (i, 0)),
      ),
      out_shape=(jax.ShapeDtypeStruct((_N, _DH), jnp.float32),
                 jax.ShapeDtypeStruct((_N, _DH), jnp.float32)),
  )(x, w)


def _bn_relu(s0_ref, s1_ref, dinv_ref, b_ref, g_ref, bt_ref):
  u = jnp.concatenate([s0_ref[: _N, :], s1_ref[: _N, :]], axis=1)
  u = u * dinv_ref[...] + b_ref[...]
  mean = jnp.mean(u, axis=0, keepdims=True)
  var = jnp.mean((u - mean) ** 2, axis=0, keepdims=True)
  h = (u - mean) * lax.rsqrt(var + 1e-5) * g_ref[...] + bt_ref[...]
  return jnp.maximum(h, 0.0)


def _tc_bn_mm_body(s0_ref, s1_ref, dinv_ref, b_ref, g_ref, bt_ref, w_ref,
                   o0_ref, o1_ref):
  h = _bn_relu(s0_ref, s1_ref, dinv_ref, b_ref, g_ref, bt_ref)
  t = jnp.dot(h, w_ref[...], preferred_element_type=jnp.float32)
  o0_ref[...] = t[:, :_DH]
  o1_ref[...] = t[:, _DH:]


@jax.jit
def _tc_bn_mm(s0, s1, dinv, b, g, bt, w):
  return pl.pallas_call(
      _tc_bn_mm_body,
      out_shape=(jax.ShapeDtypeStruct((_N, _DH), jnp.float32),
                 jax.ShapeDtypeStruct((_N, _DH), jnp.float32)),
  )(s0, s1, dinv, b, g, bt, w)


def _tc_head_body(s0_ref, s1_ref, dinv_ref, b_ref, g_ref, bt_ref,
                  w1_ref, b1_ref, w2_ref, b2_ref, w3_ref, b3_ref, o_ref):
  h = _bn_relu(s0_ref, s1_ref, dinv_ref, b_ref, g_ref, bt_ref)
  h = jnp.maximum(
      jnp.dot(h, w1_ref[...], preferred_element_type=jnp.float32)
      + b1_ref[...], 0.0)
  h = jnp.maximum(
      jnp.dot(h, w2_ref[...], preferred_element_type=jnp.float32)
      + b2_ref[...], 0.0)
  o_ref[...] = (
      jnp.dot(h, w3_ref[...], preferred_element_type=jnp.float32)
      + b3_ref[...])


@jax.jit
def _tc_head(s0, s1, dinv, b, g, bt, w1, b1, w2, b2, w3, b3):
  return pl.pallas_call(
      _tc_head_body,
      out_shape=jax.ShapeDtypeStruct((_N, 1), jnp.float32),
  )(s0, s1, dinv, b, g, bt, w1, b1, w2, b2, w3, b3)


def kernel(x, conv_W, conv_b, bn_gamma, bn_beta, head_W1, head_b1, head_W2,
           head_b2, head_W3, head_b3, edge_index, batch):
  src = edge_index[0]
  dst = edge_index[1]
  npad = _E_PAD - _E
  pad_i = jnp.arange(npad, dtype=jnp.int32)
  src_p = jnp.concatenate([src, (pad_i * 97) % _N])
  dst_p = jnp.concatenate([dst, _N + (pad_i % (_N_PAD - _N))])
  src2d = src_p.reshape(_E_PAD // 128, 128)
  dst2d = dst_p.reshape(_E_PAD // 128, 128)

  dinv = _sc_deg(dst2d)[: _N].reshape(_N, 1)

  b2 = lambda v: v.reshape(1, -1)
  t0, t1 = _tc_mm0(x, conv_W[0])
  for l in range(_L):
    s0, s1 = _sc_segsum(t0, t1, src2d, dst2d)
    if l < _L - 1:
      t0, t1 = _tc_bn_mm(s0, s1, dinv, b2(conv_b[l]), b2(bn_gamma[l]),
                         b2(bn_beta[l]), conv_W[l + 1])
    else:
      out = _tc_head(s0, s1, dinv, b2(conv_b[l]), b2(bn_gamma[l]),
                     b2(bn_beta[l]), head_W1, b2(head_b1), head_W2,
                     b2(head_b2), head_W3, b2(head_b3))
  return out


# speedup vs baseline: 5.8325x; 5.8325x over previous
import jax
import jax.numpy as jnp
from jax import lax
from jax.experimental import pallas as pl
from jax.experimental.pallas import tpu as pltpu
from jax.experimental.pallas import tpu_sc as plsc

_N = 10000
_E = 160000
_D = 256
_DH = 128
_L = 6
_N_PAD = 10240
_E_PAD = 163840
_H = 256
_EPT = _E_PAD // 16
_CH = 256
_RPT = _N_PAD // 16


def _seg_mesh():
  return plsc.VectorSubcoreMesh(core_axis_name="c", subcore_axis_name="s")


def _sc_deg_body(dst_hbm, out_hbm, hist, ebuf, rbuf, obuf, stage):
  c = lax.axis_index("c")
  sid = lax.axis_index("s")
  zeros16 = jnp.zeros((16,), jnp.float32)
  ones16 = jnp.ones((16,), jnp.float32)

  @pl.when(c == 0)
  def _():
    def zloop(i, _):
      hist[pl.ds(i * 16, 16)] = zeros16
      return ()
    lax.fori_loop(0, _N_PAD // 16, zloop, ())

    n_load = _EPT // 2048
    def chunk(k, _):
      pltpu.sync_copy(dst_hbm.at[pl.ds(sid * (_EPT // 128) + k * 16, 16)],
                      ebuf)
      def grp(g, _):
        idx = ebuf[g // 8, pl.ds((g % 8) * 16, 16)]
        plsc.addupdate_scatter(hist, [idx], ones16)
        return ()
      lax.fori_loop(0, 128, grp, ())
      return ()
    lax.fori_loop(0, n_load, chunk, ())

    pltpu.sync_copy(hist, stage.at[sid])
    plsc.subcore_barrier()

    base = sid * _RPT
    for r in range(16):
      pltpu.sync_copy(stage.at[r, pl.ds(base, _RPT)], rbuf.at[r])

    def rloop(g, _):
      v = rbuf[0, pl.ds(g * 16, 16)]
      for r in range(1, 16):
        v = v + rbuf[r, pl.ds(g * 16, 16)]
      obuf[pl.ds(g * 16, 16)] = jnp.maximum(v, 1.0)
      return ()
    lax.fori_loop(0, _RPT // 16, rloop, ())
    pltpu.sync_copy(obuf, out_hbm.at[pl.ds(base, _RPT)])


@jax.jit
def _sc_deg(dst2d):
  return pl.kernel(
      _sc_deg_body,
      out_type=jax.ShapeDtypeStruct((_N_PAD,), jnp.float32),
      mesh=_seg_mesh(),
      compiler_params=pltpu.CompilerParams(needs_layout_passes=False),
      scratch_types=[
          pltpu.VMEM((_N_PAD,), jnp.float32),
          pltpu.VMEM((16, 128), jnp.int32),
          pltpu.VMEM((16, _RPT), jnp.float32),
          pltpu.VMEM((_RPT,), jnp.float32),
          pltpu.VMEM_SHARED((16, _N_PAD), jnp.float32),
      ],
  )(dst2d)


def _sc_segsum_body(t0_hbm, t1_hbm, src_hbm, dst_hbm, s0_hbm, s1_hbm,
                    acc, rows, sidx, didx, sem):
  c = lax.axis_index("c")
  sid = lax.axis_index("s")
  zeros16 = jnp.zeros((16,), jnp.float32)

  def zloop(i, _):
    for j in range(8):
      rows[i, pl.ds(j * 16, 16)] = zeros16
    return ()
  lax.fori_loop(0, _CH, zloop, ())
  pltpu.sync_copy(rows, acc.at[pl.ds(sid * _RPT, _CH)])
  pltpu.sync_copy(rows, acc.at[pl.ds(sid * _RPT + _CH, _CH)])
  pltpu.sync_copy(rows.at[pl.ds(0, _RPT - 2 * _CH)],
                  acc.at[pl.ds(sid * _RPT + 2 * _CH, _RPT - 2 * _CH)])
  plsc.subcore_barrier()

  def edge_pass(t_hbm):
    def chunk(k, _):
      rowbase = pl.multiple_of((sid * _EPT + k * 1024) // 128, 8)
      pltpu.sync_copy(src_hbm.at[pl.ds(rowbase, 8)], sidx)
      pltpu.sync_copy(dst_hbm.at[pl.ds(rowbase, 8)], didx)
      for h in range(4):
        descs = [
            pltpu.async_copy(t_hbm.at[sidx.at[h * 2 + j]],
                             rows.at[pl.ds(j * 128, 128)], sem)
            for j in range(2)
        ]
        for d in descs:
          d.wait()
        for j in range(2):
          pltpu.sync_copy(rows.at[pl.ds(j * 128, 128)],
                          acc.at[didx.at[h * 2 + j]], add=True)
      return ()
    lax.fori_loop(0, _EPT // 1024, chunk, ())

  @pl.when(c == 0)
  def _():
    edge_pass(t0_hbm)

  @pl.when(c == 1)
  def _():
    edge_pass(t1_hbm)

  plsc.subcore_barrier()

  @pl.when(c == 0)
  def _():
    pltpu.sync_copy(acc.at[pl.ds(sid * _RPT, _RPT)],
                    s0_hbm.at[pl.ds(sid * _RPT, _RPT)])

  @pl.when(c == 1)
  def _():
    pltpu.sync_copy(acc.at[pl.ds(sid * _RPT, _RPT)],
                    s1_hbm.at[pl.ds(sid * _RPT, _RPT)])


@jax.jit
def _sc_segsum(t0, t1, src2d, dst2d):
  return pl.kernel(
      _sc_segsum_body,
      out_type=(jax.ShapeDtypeStruct((_N_PAD, _DH), jnp.float32),
                jax.ShapeDtypeStruct((_N_PAD, _DH), jnp.float32)),
      mesh=_seg_mesh(),
      compiler_params=pltpu.CompilerParams(needs_layout_passes=False),
      scratch_types=[
          pltpu.VMEM_SHARED((_N_PAD, _DH), jnp.float32),
          pltpu.VMEM((_CH, _DH), jnp.float32),
          pltpu.VMEM((8, 128), jnp.int32),
          pltpu.VMEM((8, 128), jnp.int32),
          pltpu.SemaphoreType.DMA,
      ],
  )(t0, t1, src2d, dst2d)


_BLK = 2000
_NBLK = _N // _BLK


def _tc_mm_body(s0_ref, s1_ref, deg_ref, b_ref, w_ref, o_ref):
  agg = jnp.concatenate([s0_ref[...], s1_ref[...]], axis=1) / deg_ref[...]
  o_ref[...] = (
      jnp.dot(agg, w_ref[...], preferred_element_type=jnp.float32)
      + b_ref[...])


@jax.jit
def _tc_mm(s0, s1, deg, b, w):
  return pl.pallas_call(
      _tc_mm_body,
      grid=(_NBLK,),
      in_specs=[
          pl.BlockSpec((_BLK, _DH), lambda i: (i, 0)),
          pl.BlockSpec((_BLK, _DH), lambda i: (i, 0)),
          pl.BlockSpec((_BLK, 1), lambda i: (i, 0)),
          pl.BlockSpec((1, _D), lambda i: (0, 0)),
          pl.BlockSpec((_D, _D), lambda i: (0, 0)),
      ],
      out_specs=pl.BlockSpec((_BLK, _D), lambda i: (i, 0)),
      out_shape=jax.ShapeDtypeStruct((_N, _D), jnp.float32),
  )(s0, s1, deg, b, w)


def _tc_stats_body(h_ref, o_ref, acc, mean_s):
  p = pl.program_id(0)
  i = pl.program_id(1)
  x = h_ref[...]

  @pl.when(p == 0)
  def _():
    blk = jnp.sum(x, axis=0, keepdims=True)

    @pl.when(i == 0)
    def _():
      acc[...] = blk

    @pl.when(i > 0)
    def _():
      acc[...] = acc[...] + blk

    @pl.when(i == _NBLK - 1)
    def _():
      mean_s[...] = acc[...] / _N

  @pl.when(p == 1)
  def _():
    d = x - mean_s[...]
    blk = jnp.sum(d * d, axis=0, keepdims=True)

    @pl.when(i == 0)
    def _():
      acc[...] = blk

    @pl.when(i > 0)
    def _():
      acc[...] = acc[...] + blk

    @pl.when(i == _NBLK - 1)
    def _():
      o_ref[...] = jnp.concatenate([mean_s[...], acc[...] / _N], axis=0)


@jax.jit
def _tc_stats(h):
  return pl.pallas_call(
      _tc_stats_body,
      grid=(2, _NBLK),
      in_specs=[pl.BlockSpec((_BLK, _D), lambda p, i: (i, 0))],
      out_specs=pl.BlockSpec((2, _D), lambda p, i: (0, 0)),
      out_shape=jax.ShapeDtypeStruct((2, _D), jnp.float32),
      scratch_shapes=[pltpu.VMEM((1, _D), jnp.float32),
                      pltpu.VMEM((1, _D), jnp.float32)],
  )(h)


def _bn_relu_block(h_ref, st_ref, g_ref, bt_ref):
  x = h_ref[...]
  mean = st_ref[0:1, :]
  var = st_ref[1:2, :]
  return jnp.maximum(
      (x - mean) / jnp.sqrt(var + 1e-5) * g_ref[...] + bt_ref[...], 0.0)


def _tc_apply_body(h_ref, st_ref, g_ref, bt_ref, o0_ref, o1_ref):
  h = _bn_relu_block(h_ref, st_ref, g_ref, bt_ref)
  o0_ref[...] = h[:, :_DH]
  o1_ref[...] = h[:, _DH:]


@jax.jit
def _tc_apply(h, st, g, bt):
  return pl.pallas_call(
      _tc_apply_body,
      grid=(_NBLK,),
      in_specs=[
          pl.BlockSpec((_BLK, _D), lambda i: (i, 0)),
          pl.BlockSpec((2, _D), lambda i: (0, 0)),
          pl.BlockSpec((1, _D), lambda i: (0, 0)),
          pl.BlockSpec((1, _D), lambda i: (0, 0)),
      ],
      out_specs=(
          pl.BlockSpec((_BLK, _DH), lambda i: (i, 0)),
          pl.BlockSpec((_BLK, _DH), lambda i: (i, 0)),
      ),
      out_shape=(jax.ShapeDtypeStruct((_N, _DH), jnp.float32),
                 jax.ShapeDtypeStruct((_N, _DH), jnp.float32)),
  )(h, st, g, bt)


def _tc_head_body(h_ref, st_ref, g_ref, bt_ref, w1_ref, b1_ref, w2_ref,
                  b2_ref, w3_ref, b3_ref, o_ref):
  h = _bn_relu_block(h_ref, st_ref, g_ref, bt_ref)
  h = jnp.maximum(
      jnp.dot(h, w1_ref[...], preferred_element_type=jnp.float32)
      + b1_ref[...], 0.0)
  h = jnp.maximum(
      jnp.dot(h, w2_ref[...], preferred_element_type=jnp.float32)
      + b2_ref[...], 0.0)
  o_ref[...] = (
      jnp.dot(h, w3_ref[...], preferred_element_type=jnp.float32)
      + b3_ref[...])


@jax.jit
def _tc_head(h, st, g, bt, w1, b1, w2, b2, w3, b3):
  return pl.pallas_call(
      _tc_head_body,
      grid=(_NBLK,),
      in_specs=[
          pl.BlockSpec((_BLK, _D), lambda i: (i, 0)),
          pl.BlockSpec((2, _D), lambda i: (0, 0)),
          pl.BlockSpec((1, _D), lambda i: (0, 0)),
          pl.BlockSpec((1, _D), lambda i: (0, 0)),
          pl.BlockSpec((_D, _H), lambda i: (0, 0)),
          pl.BlockSpec((1, _H), lambda i: (0, 0)),
          pl.BlockSpec((_H, _H), lambda i: (0, 0)),
          pl.BlockSpec((1, _H), lambda i: (0, 0)),
          pl.BlockSpec((_H, 1), lambda i: (0, 0)),
          pl.BlockSpec((1, 1), lambda i: (0, 0)),
      ],
      out_specs=pl.BlockSpec((_BLK, 1), lambda i: (i, 0)),
      out_shape=jax.ShapeDtypeStruct((_N, 1), jnp.float32),
  )(h, st, g, bt, w1, b1, w2, b2, w3, b3)


def kernel(x, conv_W, conv_b, bn_gamma, bn_beta, head_W1, head_b1, head_W2,
           head_b2, head_W3, head_b3, edge_index, batch):
  src = edge_index[0]
  dst = edge_index[1]
  npad = _E_PAD - _E
  pad_i = jnp.arange(npad, dtype=jnp.int32)
  src_p = jnp.concatenate([src, (pad_i * 97) % _N])
  dst_p = jnp.concatenate([dst, _N + (pad_i % (_N_PAD - _N))])
  src2d = src_p.reshape(_E_PAD // 128, 128)
  dst2d = dst_p.reshape(_E_PAD // 128, 128)

  deg = _sc_deg(dst2d)[: _N].reshape(_N, 1)

  b2 = lambda v: v.reshape(1, -1)
  h0, h1 = x[:, :_DH], x[:, _DH:]
  for l in range(_L):
    s0, s1 = _sc_segsum(h0, h1, src2d, dst2d)
    hpre = _tc_mm(s0, s1, deg, b2(conv_b[l]), conv_W[l])
    st = _tc_stats(hpre)
    if l < _L - 1:
      h0, h1 = _tc_apply(hpre, st, b2(bn_gamma[l]), b2(bn_beta[l]))
    else:
      out = _tc_head(hpre, st, b2(bn_gamma[l]), b2(bn_beta[l]),
                     head_W1, b2(head_b1), head_W2, b2(head_b2),
                     head_W3, b2(head_b3))
  return out


# speedup vs baseline: 5.9552x; 1.0210x over previous
import jax
import jax.numpy as jnp
from jax import lax
from jax.experimental import pallas as pl
from jax.experimental.pallas import tpu as pltpu
from jax.experimental.pallas import tpu_sc as plsc

_N = 10000
_E = 160000
_D = 256
_DH = 128
_L = 6
_N_PAD = 10240
_E_PAD = 163840
_H = 256
_EPT = _E_PAD // 16
_CH = 256
_RPT = _N_PAD // 16


def _seg_mesh():
  return plsc.VectorSubcoreMesh(core_axis_name="c", subcore_axis_name="s")


def _sc_deg_body(dst_hbm, out_hbm, hist, ebuf, rbuf, obuf, stage):
  c = lax.axis_index("c")
  sid = lax.axis_index("s")
  zeros16 = jnp.zeros((16,), jnp.float32)
  ones16 = jnp.ones((16,), jnp.float32)

  @pl.when(c == 0)
  def _():
    def zloop(i, _):
      hist[pl.ds(i * 16, 16)] = zeros16
      return ()
    lax.fori_loop(0, _N_PAD // 16, zloop, ())

    n_load = _EPT // 2048
    def chunk(k, _):
      pltpu.sync_copy(dst_hbm.at[pl.ds(sid * (_EPT // 128) + k * 16, 16)],
                      ebuf)
      def grp(g, _):
        idx = ebuf[g // 8, pl.ds((g % 8) * 16, 16)]
        plsc.addupdate_scatter(hist, [idx], ones16)
        return ()
      lax.fori_loop(0, 128, grp, ())
      return ()
    lax.fori_loop(0, n_load, chunk, ())

    pltpu.sync_copy(hist, stage.at[sid])
    plsc.subcore_barrier()

    base = sid * _RPT
    for r in range(16):
      pltpu.sync_copy(stage.at[r, pl.ds(base, _RPT)], rbuf.at[r])

    def rloop(g, _):
      v = rbuf[0, pl.ds(g * 16, 16)]
      for r in range(1, 16):
        v = v + rbuf[r, pl.ds(g * 16, 16)]
      obuf[pl.ds(g * 16, 16)] = jnp.maximum(v, 1.0)
      return ()
    lax.fori_loop(0, _RPT // 16, rloop, ())
    pltpu.sync_copy(obuf, out_hbm.at[pl.ds(base, _RPT)])


@jax.jit
def _sc_deg(dst2d):
  return pl.kernel(
      _sc_deg_body,
      out_type=jax.ShapeDtypeStruct((_N_PAD,), jnp.float32),
      mesh=_seg_mesh(),
      compiler_params=pltpu.CompilerParams(needs_layout_passes=False),
      scratch_types=[
          pltpu.VMEM((_N_PAD,), jnp.float32),
          pltpu.VMEM((16, 128), jnp.int32),
          pltpu.VMEM((16, _RPT), jnp.float32),
          pltpu.VMEM((_RPT,), jnp.float32),
          pltpu.VMEM_SHARED((16, _N_PAD), jnp.float32),
      ],
  )(dst2d)


def _sc_segsum_body(t0_hbm, t1_hbm, src_hbm, dst_hbm, s0_hbm, s1_hbm,
                    acc, rows, sidx, didx, sem):
  c = lax.axis_index("c")
  sid = lax.axis_index("s")
  zeros16 = jnp.zeros((16,), jnp.float32)

  def zloop(i, _):
    for j in range(8):
      rows[i, pl.ds(j * 16, 16)] = zeros16
    return ()
  lax.fori_loop(0, _CH, zloop, ())
  pltpu.sync_copy(rows, acc.at[pl.ds(sid * _RPT, _CH)])
  pltpu.sync_copy(rows, acc.at[pl.ds(sid * _RPT + _CH, _CH)])
  pltpu.sync_copy(rows.at[pl.ds(0, _RPT - 2 * _CH)],
                  acc.at[pl.ds(sid * _RPT + 2 * _CH, _RPT - 2 * _CH)])
  plsc.subcore_barrier()

  def edge_pass(t_hbm):
    def chunk(k, _):
      rowbase = pl.multiple_of((sid * _EPT + k * 1024) // 128, 8)
      pltpu.sync_copy(src_hbm.at[pl.ds(rowbase, 8)], sidx)
      pltpu.sync_copy(dst_hbm.at[pl.ds(rowbase, 8)], didx)
      for h in range(4):
        descs = [
            pltpu.async_copy(t_hbm.at[sidx.at[h * 2 + j]],
                             rows.at[pl.ds(j * 128, 128)], sem)
            for j in range(2)
        ]
        for d in descs:
          d.wait()
        for j in range(2):
          pltpu.sync_copy(rows.at[pl.ds(j * 128, 128)],
                          acc.at[didx.at[h * 2 + j]], add=True)
      return ()
    lax.fori_loop(0, _EPT // 1024, chunk, ())

  @pl.when(c == 0)
  def _():
    edge_pass(t0_hbm)

  @pl.when(c == 1)
  def _():
    edge_pass(t1_hbm)

  plsc.subcore_barrier()

  @pl.when(c == 0)
  def _():
    pltpu.sync_copy(acc.at[pl.ds(sid * _RPT, _RPT)],
                    s0_hbm.at[pl.ds(sid * _RPT, _RPT)])

  @pl.when(c == 1)
  def _():
    pltpu.sync_copy(acc.at[pl.ds(sid * _RPT, _RPT)],
                    s1_hbm.at[pl.ds(sid * _RPT, _RPT)])


@jax.jit
def _sc_segsum(t0, t1, src2d, dst2d):
  return pl.kernel(
      _sc_segsum_body,
      out_type=(jax.ShapeDtypeStruct((_N_PAD, _DH), jnp.float32),
                jax.ShapeDtypeStruct((_N_PAD, _DH), jnp.float32)),
      mesh=_seg_mesh(),
      compiler_params=pltpu.CompilerParams(needs_layout_passes=False),
      scratch_types=[
          pltpu.VMEM_SHARED((_N_PAD, _DH), jnp.float32),
          pltpu.VMEM((_CH, _DH), jnp.float32),
          pltpu.VMEM((8, 128), jnp.int32),
          pltpu.VMEM((8, 128), jnp.int32),
          pltpu.SemaphoreType.DMA,
      ],
  )(t0, t1, src2d, dst2d)


_BLK = 2000
_NBLK = _N // _BLK


def _tc_mm_body(s0_ref, s1_ref, deg_ref, b_ref, w_ref, o_ref):
  agg = jnp.concatenate([s0_ref[...], s1_ref[...]], axis=1) / deg_ref[...]
  o_ref[...] = (
      jnp.dot(agg, w_ref[...], preferred_element_type=jnp.float32)
      + b_ref[...])


@jax.jit
def _tc_mm(s0, s1, deg, b, w):
  return pl.pallas_call(
      _tc_mm_body,
      grid=(_NBLK,),
      in_specs=[
          pl.BlockSpec((_BLK, _DH), lambda i: (i, 0)),
          pl.BlockSpec((_BLK, _DH), lambda i: (i, 0)),
          pl.BlockSpec((_BLK, 1), lambda i: (i, 0)),
          pl.BlockSpec((1, _D), lambda i: (0, 0)),
          pl.BlockSpec((_D, _D), lambda i: (0, 0)),
      ],
      out_specs=pl.BlockSpec((_BLK, _D), lambda i: (i, 0)),
      out_shape=jax.ShapeDtypeStruct((_N, _D), jnp.float32),
  )(s0, s1, deg, b, w)


def _tc_stats_body(h_ref, o_ref):
  x = h_ref[...]
  mean = jnp.mean(x, axis=0, keepdims=True)
  var = jnp.var(x, axis=0, keepdims=True)
  o_ref[...] = jnp.concatenate([mean, var], axis=0)


@jax.jit
def _tc_stats(h):
  return pl.pallas_call(
      _tc_stats_body,
      out_shape=jax.ShapeDtypeStruct((2, _D), jnp.float32),
  )(h)


def _bn_relu_block(h_ref, st_ref, g_ref, bt_ref):
  x = h_ref[...]
  mean = st_ref[0:1, :]
  var = st_ref[1:2, :]
  return jnp.maximum(
      (x - mean) / jnp.sqrt(var + 1e-5) * g_ref[...] + bt_ref[...], 0.0)


def _tc_apply_body(h_ref, st_ref, g_ref, bt_ref, o0_ref, o1_ref):
  h = _bn_relu_block(h_ref, st_ref, g_ref, bt_ref)
  o0_ref[...] = h[:, :_DH]
  o1_ref[...] = h[:, _DH:]


@jax.jit
def _tc_apply(h, st, g, bt):
  return pl.pallas_call(
      _tc_apply_body,
      grid=(_NBLK,),
      in_specs=[
          pl.BlockSpec((_BLK, _D), lambda i: (i, 0)),
          pl.BlockSpec((2, _D), lambda i: (0, 0)),
          pl.BlockSpec((1, _D), lambda i: (0, 0)),
          pl.BlockSpec((1, _D), lambda i: (0, 0)),
      ],
      out_specs=(
          pl.BlockSpec((_BLK, _DH), lambda i: (i, 0)),
          pl.BlockSpec((_BLK, _DH), lambda i: (i, 0)),
      ),
      out_shape=(jax.ShapeDtypeStruct((_N, _DH), jnp.float32),
                 jax.ShapeDtypeStruct((_N, _DH), jnp.float32)),
  )(h, st, g, bt)


def _tc_head_body(h_ref, st_ref, g_ref, bt_ref, w1_ref, b1_ref, w2_ref,
                  b2_ref, w3_ref, b3_ref, o_ref):
  h = _bn_relu_block(h_ref, st_ref, g_ref, bt_ref)
  h = jnp.maximum(
      jnp.dot(h, w1_ref[...], preferred_element_type=jnp.float32)
      + b1_ref[...], 0.0)
  h = jnp.maximum(
      jnp.dot(h, w2_ref[...], preferred_element_type=jnp.float32)
      + b2_ref[...], 0.0)
  o_ref[...] = (
      jnp.dot(h, w3_ref[...], preferred_element_type=jnp.float32)
      + b3_ref[...])


@jax.jit
def _tc_head(h, st, g, bt, w1, b1, w2, b2, w3, b3):
  return pl.pallas_call(
      _tc_head_body,
      grid=(_NBLK,),
      in_specs=[
          pl.BlockSpec((_BLK, _D), lambda i: (i, 0)),
          pl.BlockSpec((2, _D), lambda i: (0, 0)),
          pl.BlockSpec((1, _D), lambda i: (0, 0)),
          pl.BlockSpec((1, _D), lambda i: (0, 0)),
          pl.BlockSpec((_D, _H), lambda i: (0, 0)),
          pl.BlockSpec((1, _H), lambda i: (0, 0)),
          pl.BlockSpec((_H, _H), lambda i: (0, 0)),
          pl.BlockSpec((1, _H), lambda i: (0, 0)),
          pl.BlockSpec((_H, 1), lambda i: (0, 0)),
          pl.BlockSpec((1, 1), lambda i: (0, 0)),
      ],
      out_specs=pl.BlockSpec((_BLK, 1), lambda i: (i, 0)),
      out_shape=jax.ShapeDtypeStruct((_N, 1), jnp.float32),
  )(h, st, g, bt, w1, b1, w2, b2, w3, b3)


def kernel(x, conv_W, conv_b, bn_gamma, bn_beta, head_W1, head_b1, head_W2,
           head_b2, head_W3, head_b3, edge_index, batch):
  src = edge_index[0]
  dst = edge_index[1]
  npad = _E_PAD - _E
  pad_i = jnp.arange(npad, dtype=jnp.int32)
  src_p = jnp.concatenate([src, (pad_i * 97) % _N])
  dst_p = jnp.concatenate([dst, _N + (pad_i % (_N_PAD - _N))])
  src2d = src_p.reshape(_E_PAD // 128, 128)
  dst2d = dst_p.reshape(_E_PAD // 128, 128)

  deg = _sc_deg(dst2d)[: _N].reshape(_N, 1)

  b2 = lambda v: v.reshape(1, -1)
  h0, h1 = x[:, :_DH], x[:, _DH:]
  for l in range(_L):
    s0, s1 = _sc_segsum(h0, h1, src2d, dst2d)
    hpre = _tc_mm(s0, s1, deg, b2(conv_b[l]), conv_W[l])
    st = _tc_stats(hpre)
    if l < _L - 1:
      h0, h1 = _tc_apply(hpre, st, b2(bn_gamma[l]), b2(bn_beta[l]))
    else:
      out = _tc_head(hpre, st, b2(bn_gamma[l]), b2(bn_beta[l]),
                     head_W1, b2(head_b1), head_W2, b2(head_b2),
                     head_W3, b2(head_b3))
  return out
